# Initial kernel scaffold; baseline (speedup 1.0000x reference)
import jax
import jax.numpy as jnp
from jax.experimental import pallas as pl


def kernel(x):
    raise NotImplementedError("write your pallas kernel here")


# speedup vs baseline: 181.9147x; 181.9147x over previous
import jax
import jax.numpy as jnp
from jax.experimental import pallas as pl
from jax.experimental.pallas import tpu as pltpu

_T = 16
_BLK = 256


def _minmax_body(x_ref, min_ref, max_ref):
    min_ref[0, 0] = jnp.min(x_ref[...])
    max_ref[0, 0] = jnp.max(x_ref[...])


def _encode_body(min_ref, max_ref, x_ref, out_ref):
    mn = min_ref[0, 0]
    mx = max_ref[0, 0]
    x = x_ref[...]
    xn = jnp.clip((x - mn) / (mx - mn + 1e-8), 0.0, 1.0)
    lat = ((1.0 - xn) * (_T - 1)).astype(jnp.int32)
    t = jax.lax.broadcasted_iota(jnp.int32, (x.shape[0], _T, x.shape[1]), 1)
    out_ref[...] = (lat[:, None, :] == t).astype(jnp.float32)


def kernel(x):
    B, F = x.shape
    mn, mx = pl.pallas_call(
        _minmax_body,
        out_shape=(
            jax.ShapeDtypeStruct((1, 1), jnp.float32),
            jax.ShapeDtypeStruct((1, 1), jnp.float32),
        ),
        out_specs=(
            pl.BlockSpec(memory_space=pltpu.SMEM),
            pl.BlockSpec(memory_space=pltpu.SMEM),
        ),
    )(x)

    spikes = pl.pallas_call(
        _encode_body,
        grid=(B // _BLK,),
        in_specs=(
            pl.BlockSpec(memory_space=pltpu.SMEM),
            pl.BlockSpec(memory_space=pltpu.SMEM),
            pl.BlockSpec((_BLK, F), lambda i: (i, 0)),
        ),
        out_specs=pl.BlockSpec((_BLK, _T, F), lambda i: (i, 0, 0)),
        out_shape=jax.ShapeDtypeStruct((B, _T, F), jnp.float32),
    )(mn, mx, x)
    return spikes
